# baseline (speedup 1.0000x reference)
import jax
import jax.numpy as jnp
from jax.experimental import pallas as pl


def _add_one_body(x_ref, y_ref, pk_ref):
    x = x_ref[...]
    y_ref[...] = x + 1.0
    pk_ref[...] = x


def kernel(key_states, token_idx, param):
    S, BS, D0, D1 = key_states.shape
    rows, cols = S * BS, D0 * D1
    x2 = key_states.reshape(rows, cols)
    R = 256
    y2, pk2 = pl.pallas_call(
        _add_one_body,
        grid=(rows // R,),
        in_specs=[pl.BlockSpec((R, cols), lambda i: (i, 0))],
        out_specs=[
            pl.BlockSpec((R, cols), lambda i: (i, 0)),
            pl.BlockSpec((R, cols), lambda i: (i, 0)),
        ],
        out_shape=[
            jax.ShapeDtypeStruct((rows, cols), key_states.dtype),
            jax.ShapeDtypeStruct((rows, cols), key_states.dtype),
        ],
    )(x2)
    return y2.reshape(key_states.shape), pk2.reshape(key_states.shape)


# speedup vs baseline: 1.0315x; 1.0315x over previous
import jax
import jax.numpy as jnp
from jax.experimental import pallas as pl


def _add_one_body(x_ref, y_ref, pk_ref):
    x = x_ref[...]
    y_ref[...] = x + 1.0
    pk_ref[...] = x


def kernel(key_states, token_idx, param):
    S, BS, D0, D1 = key_states.shape
    R = 16
    spec = pl.BlockSpec((R, BS, D0, D1), lambda i: (i, 0, 0, 0))
    y, pk = pl.pallas_call(
        _add_one_body,
        grid=(S // R,),
        in_specs=[spec],
        out_specs=[spec, spec],
        out_shape=[
            jax.ShapeDtypeStruct((S, BS, D0, D1), key_states.dtype),
            jax.ShapeDtypeStruct((S, BS, D0, D1), key_states.dtype),
        ],
    )(key_states)
    return y, pk


# speedup vs baseline: 11.3467x; 10.9998x over previous
import jax
import jax.numpy as jnp
from jax.experimental import pallas as pl


def _add_one_body(x_ref, y_ref, pk_ref):
    x = x_ref[...]
    y_ref[...] = x + 1.0
    pk_ref[...] = x


def kernel(key_states, token_idx, param):
    S, BS, D0, D1 = key_states.shape
    xt = jnp.swapaxes(key_states, 2, 3)
    R = 32
    spec = pl.BlockSpec((R, BS, D1, D0), lambda i: (i, 0, 0, 0))
    y, pk = pl.pallas_call(
        _add_one_body,
        grid=(S // R,),
        in_specs=[spec],
        out_specs=[spec, spec],
        out_shape=[
            jax.ShapeDtypeStruct((S, BS, D1, D0), key_states.dtype),
            jax.ShapeDtypeStruct((S, BS, D1, D0), key_states.dtype),
        ],
    )(xt)
    return jnp.swapaxes(y, 2, 3), jnp.swapaxes(pk, 2, 3)


# speedup vs baseline: 11.6190x; 1.0240x over previous
import jax
import jax.numpy as jnp
from jax.experimental import pallas as pl


def _add_one_body(x_ref, y_ref, pk_ref):
    x = x_ref[...]
    y_ref[...] = x + 1.0
    pk_ref[...] = x


def kernel(key_states, token_idx, param):
    S, BS, D0, D1 = key_states.shape
    xt = jnp.swapaxes(key_states, 2, 3)
    R = 64
    spec = pl.BlockSpec((R, BS, D1, D0), lambda i: (i, 0, 0, 0))
    y, pk = pl.pallas_call(
        _add_one_body,
        grid=(S // R,),
        in_specs=[spec],
        out_specs=[spec, spec],
        out_shape=[
            jax.ShapeDtypeStruct((S, BS, D1, D0), key_states.dtype),
            jax.ShapeDtypeStruct((S, BS, D1, D0), key_states.dtype),
        ],
    )(xt)
    return jnp.swapaxes(y, 2, 3), jnp.swapaxes(pk, 2, 3)


# speedup vs baseline: 11.6380x; 1.0016x over previous
import jax
import jax.numpy as jnp
from jax.experimental import pallas as pl


def _add_one_body(x_ref, y_ref, pk_ref):
    y_ref[...] = x_ref[...] + 1.0
    pk_ref[...] = x_ref[...]


def kernel(key_states, token_idx, param):
    S, BS, D0, D1 = key_states.shape
    xt = jnp.swapaxes(key_states, 2, 3)
    R = 64
    spec = pl.BlockSpec((R, BS, D1, D0), lambda i: (i, 0, 0, 0))
    y, pk = pl.pallas_call(
        _add_one_body,
        grid=(S // R,),
        in_specs=[spec],
        out_specs=[spec, spec],
        out_shape=[
            jax.ShapeDtypeStruct((S, BS, D1, D0), key_states.dtype),
            jax.ShapeDtypeStruct((S, BS, D1, D0), key_states.dtype),
        ],
    )(xt)
    return jnp.swapaxes(y, 2, 3), jnp.swapaxes(pk, 2, 3)
